# baseline (speedup 1.0000x reference)
import jax
import jax.numpy as jnp
from jax.experimental import pallas as pl
from jax.experimental.pallas import tpu as pltpu

_H = 16
_D = 64
_K = 1024
_CHUNK = 2048


def _vq_kernel(a_ref, x_ref, o_ref):
    a = a_ref[0]
    xc = x_ref[0]
    s = jax.lax.dot_general(
        a, xc, (((1,), (0,)), ((), ())),
        preferred_element_type=jnp.float32)
    m_sq = jnp.sum(a * a, axis=1, keepdims=True) * 0.25
    d = s + m_sq
    o_ref[0, 0, 0, :] = jnp.argmin(d, axis=0).astype(jnp.int32)


def _half(a, xt, bn):
    nc = bn // _CHUNK
    return pl.pallas_call(
        _vq_kernel,
        grid=(_H, nc),
        in_specs=[
            pl.BlockSpec((1, _K, _D), lambda hh, c: (hh, 0, 0)),
            pl.BlockSpec((1, _D, _CHUNK), lambda hh, c: (hh, 0, c)),
        ],
        out_specs=pl.BlockSpec((1, 1, 1, _CHUNK), lambda hh, c: (hh, c, 0, 0)),
        out_shape=jax.ShapeDtypeStruct((_H, nc, 1, _CHUNK), jnp.int32),
        compiler_params=pltpu.CompilerParams(
            dimension_semantics=("parallel", "parallel")),
    )(a, xt)


def kernel(x, means):
    b, n, feat = x.shape
    bn = b * n
    h, k, dim = means.shape
    a = -2.0 * means
    half = bn // 2
    x3 = x.reshape(bn, h, dim)
    xt0 = x3[:half].transpose(1, 2, 0)
    xt1 = x3[half:].transpose(1, 2, 0)
    o0 = _half(a, xt0, half)
    o1 = _half(a, xt1, half)
    out = jnp.concatenate([o0.reshape(_H, half), o1.reshape(_H, half)], axis=1)
    return out.reshape(_H, bn).T.reshape(b, n, _H)


# speedup vs baseline: 1.2139x; 1.2139x over previous
import jax
import jax.numpy as jnp
from jax.experimental import pallas as pl
from jax.experimental.pallas import tpu as pltpu

_H = 16
_D = 64
_K = 1024
_CHUNK = 4096


def _vq_kernel(a_ref, x_ref, o_ref):
    a = a_ref[0]
    xc = x_ref[0]
    s = jax.lax.dot_general(
        a, xc, (((1,), (0,)), ((), ())),
        preferred_element_type=jnp.float32)
    m_sq = jnp.sum(a * a, axis=1, keepdims=True) * 0.25
    d = s + m_sq
    o_ref[0, 0, 0, :] = jnp.argmin(d, axis=0).astype(jnp.int32)


def kernel(x, means):
    b, n, feat = x.shape
    bn = b * n
    h, k, dim = means.shape
    a = -2.0 * means
    xt = x.reshape(bn, h, dim).transpose(1, 2, 0)
    nc = bn // _CHUNK
    out = pl.pallas_call(
        _vq_kernel,
        grid=(_H, nc),
        in_specs=[
            pl.BlockSpec((1, _K, _D), lambda hh, c: (hh, 0, 0)),
            pl.BlockSpec((1, _D, _CHUNK), lambda hh, c: (hh, 0, c)),
        ],
        out_specs=pl.BlockSpec((1, 1, 1, _CHUNK), lambda hh, c: (hh, c, 0, 0)),
        out_shape=jax.ShapeDtypeStruct((_H, nc, 1, _CHUNK), jnp.int32),
        compiler_params=pltpu.CompilerParams(
            dimension_semantics=("parallel", "parallel")),
    )(a, xt)
    return out.reshape(_H, bn).T.reshape(b, n, _H)


# speedup vs baseline: 1.2942x; 1.0661x over previous
import jax
import jax.numpy as jnp
from jax.experimental import pallas as pl
from jax.experimental.pallas import tpu as pltpu

_H = 16
_D = 64
_K = 1024
_CHUNK = 4096


def _vq_kernel(m_ref, x_ref, o_ref):
    a = -2.0 * m_ref[0]
    xc = x_ref[0]
    s = jax.lax.dot_general(
        a, xc, (((1,), (0,)), ((), ())),
        preferred_element_type=jnp.float32)
    m = m_ref[0]
    m_sq = jnp.sum(m * m, axis=1, keepdims=True)
    d = s + m_sq
    o_ref[0, 0, 0, :] = jnp.argmin(d, axis=0).astype(jnp.int32)


def kernel(x, means):
    b, n, feat = x.shape
    bn = b * n
    h, k, dim = means.shape
    xt = x.reshape(bn, h, dim).transpose(1, 2, 0)
    nc = bn // _CHUNK
    out = pl.pallas_call(
        _vq_kernel,
        grid=(_H, nc),
        in_specs=[
            pl.BlockSpec((1, _K, _D), lambda hh, c: (hh, 0, 0)),
            pl.BlockSpec((1, _D, _CHUNK), lambda hh, c: (hh, 0, c)),
        ],
        out_specs=pl.BlockSpec((1, 1, 1, _CHUNK), lambda hh, c: (hh, c, 0, 0)),
        out_shape=jax.ShapeDtypeStruct((_H, nc, 1, _CHUNK), jnp.int32),
        compiler_params=pltpu.CompilerParams(
            dimension_semantics=("parallel", "parallel")),
    )(means, xt)
    return out.reshape(_H, bn).T.reshape(b, n, _H)
